# baseline (speedup 1.0000x reference)
import functools

import jax
import jax.numpy as jnp
from jax import lax
from jax.experimental import pallas as pl
from jax.experimental.pallas import tpu as pltpu
from jax.experimental.pallas import tpu_sc as plsc

_VOCAB_SIZE = 1000000
_UNKNOWN_TOKEN_ID = 0
_EMBED_DIM = 32

_NC = 2
_NS = 16
_L = 16
_NW = _NC * _NS

_NBUF = 2
_CHUNK = 1600


@functools.lru_cache(maxsize=None)
def _make_gather(B, D, V):
    assert B % (_NW * _NBUF * _CHUNK) == 0
    b_per_w = B // _NW
    n_chunks = b_per_w // _CHUNK
    mesh = plsc.VectorSubcoreMesh(core_axis_name="c", subcore_axis_name="s")

    scratch = (
        [pltpu.VMEM((_CHUNK,), jnp.int32) for _ in range(_NBUF)]
        + [pltpu.VMEM((_CHUNK, D), jnp.float32) for _ in range(_NBUF)]
        + [pltpu.SemaphoreType.DMA] * (3 * _NBUF)
    )

    @functools.partial(
        pl.kernel,
        mesh=mesh,
        out_type=jax.ShapeDtypeStruct((B, D), jnp.float32),
        scratch_types=scratch,
        compiler_params=pltpu.CompilerParams(use_tc_tiling_on_sc=False),
    )
    def gather_kernel(idx_hbm, table_hbm, out_hbm, *bufs):
        ib = bufs[:_NBUF]
        rows = bufs[_NBUF:2 * _NBUF]
        isem = bufs[2 * _NBUF:3 * _NBUF]
        gsem = bufs[3 * _NBUF:4 * _NBUF]
        ssem = bufs[4 * _NBUF:5 * _NBUF]

        wid = lax.axis_index("s") * _NC + lax.axis_index("c")
        base = wid * b_per_w

        def start_idx(c, b):
            pltpu.async_copy(
                idx_hbm.at[pl.ds(base + c * _CHUNK, _CHUNK)], ib[b], isem[b])

        def wait_idx(c, b):
            pltpu.make_async_copy(
                idx_hbm.at[pl.ds(base + c * _CHUNK, _CHUNK)], ib[b],
                isem[b]).wait()

        def clamp(b):
            def clamp_body(i, carry):
                v = ib[b][pl.ds(i * _L, _L)]
                ib[b][pl.ds(i * _L, _L)] = jnp.where(
                    v >= V, _UNKNOWN_TOKEN_ID, v)
                return carry
            lax.fori_loop(0, _CHUNK // _L, clamp_body, 0)

        def start_gather(b):
            pltpu.async_copy(table_hbm.at[ib[b]], rows[b], gsem[b])

        def wait_gather(b):
            pltpu.make_async_copy(
                table_hbm.at[ib[b]], rows[b], gsem[b]).wait()

        def start_store(c, b):
            pltpu.async_copy(
                rows[b], out_hbm.at[pl.ds(base + c * _CHUNK, _CHUNK)],
                ssem[b])

        def wait_store(c, b):
            pltpu.make_async_copy(
                rows[b], out_hbm.at[pl.ds(base + c * _CHUNK, _CHUNK)],
                ssem[b]).wait()

        for b in range(_NBUF):
            start_idx(b, b)
        for b in range(_NBUF):
            wait_idx(b, b)
            clamp(b)
            start_gather(b)

        def ring_body(g, carry):
            for b in range(_NBUF):
                c = g * _NBUF + b
                wait_gather(b)
                start_store(c, b)

                @pl.when(c + _NBUF < n_chunks)
                def _():
                    start_idx(c + _NBUF, b)

                wait_store(c, b)

                @pl.when(c + _NBUF < n_chunks)
                def _():
                    wait_idx(c + _NBUF, b)
                    clamp(b)
                    start_gather(b)
            return carry

        lax.fori_loop(0, n_chunks // _NBUF, ring_body, 0)

    return gather_kernel


@jax.jit
def kernel(x, table):
    batch, seq = x.shape
    flat = x.reshape(batch * seq).astype(jnp.int32)
    out = _make_gather(batch * seq, _EMBED_DIM, _VOCAB_SIZE)(flat, table)
    return out.reshape(batch, seq, _EMBED_DIM)


# speedup vs baseline: 1.0007x; 1.0007x over previous
import functools

import jax
import jax.numpy as jnp
from jax import lax
from jax.experimental import pallas as pl
from jax.experimental.pallas import tpu as pltpu
from jax.experimental.pallas import tpu_sc as plsc

_VOCAB_SIZE = 1000000
_UNKNOWN_TOKEN_ID = 0
_EMBED_DIM = 32

_NC = 2
_NS = 16
_L = 16
_NW = _NC * _NS

_NBUF = 2
_CHUNK = 1280


@functools.lru_cache(maxsize=None)
def _make_gather(B, D, V):
    assert B % (_NW * _NBUF * _CHUNK) == 0
    b_per_w = B // _NW
    n_chunks = b_per_w // _CHUNK
    mesh = plsc.VectorSubcoreMesh(core_axis_name="c", subcore_axis_name="s")

    scratch = (
        [pltpu.VMEM((_CHUNK,), jnp.int32) for _ in range(_NBUF)]
        + [pltpu.VMEM((_CHUNK, D), jnp.float32) for _ in range(_NBUF)]
        + [pltpu.SemaphoreType.DMA] * (3 * _NBUF)
    )

    @functools.partial(
        pl.kernel,
        mesh=mesh,
        out_type=jax.ShapeDtypeStruct((B, D), jnp.float32),
        scratch_types=scratch,
        compiler_params=pltpu.CompilerParams(use_tc_tiling_on_sc=False),
    )
    def gather_kernel(idx_hbm, table_hbm, out_hbm, *bufs):
        ib = bufs[:_NBUF]
        rows = bufs[_NBUF:2 * _NBUF]
        isem = bufs[2 * _NBUF:3 * _NBUF]
        gsem = bufs[3 * _NBUF:4 * _NBUF]
        ssem = bufs[4 * _NBUF:5 * _NBUF]

        wid = lax.axis_index("s") * _NC + lax.axis_index("c")
        base = wid * b_per_w

        def start_idx(c, b):
            pltpu.async_copy(
                idx_hbm.at[pl.ds(base + c * _CHUNK, _CHUNK)], ib[b], isem[b])

        def wait_idx(c, b):
            pltpu.make_async_copy(
                idx_hbm.at[pl.ds(base + c * _CHUNK, _CHUNK)], ib[b],
                isem[b]).wait()

        def clamp(b):
            def clamp_body(i, carry):
                v = ib[b][pl.ds(i * _L, _L)]
                ib[b][pl.ds(i * _L, _L)] = jnp.where(
                    v >= V, _UNKNOWN_TOKEN_ID, v)
                return carry
            lax.fori_loop(0, _CHUNK // _L, clamp_body, 0)

        def start_gather(b):
            pltpu.async_copy(table_hbm.at[ib[b]], rows[b], gsem[b])

        def wait_gather(b):
            pltpu.make_async_copy(
                table_hbm.at[ib[b]], rows[b], gsem[b]).wait()

        def start_store(c, b):
            pltpu.async_copy(
                rows[b], out_hbm.at[pl.ds(base + c * _CHUNK, _CHUNK)],
                ssem[b])

        def wait_store(c, b):
            pltpu.make_async_copy(
                rows[b], out_hbm.at[pl.ds(base + c * _CHUNK, _CHUNK)],
                ssem[b]).wait()

        for b in range(_NBUF):
            start_idx(b, b)
        for b in range(_NBUF):
            wait_idx(b, b)
            clamp(b)
            start_gather(b)

        def ring_body(g, carry):
            for b in range(_NBUF):
                c = g * _NBUF + b
                wait_gather(b)
                start_store(c, b)

                @pl.when(c + _NBUF < n_chunks)
                def _():
                    start_idx(c + _NBUF, b)

                wait_store(c, b)

                @pl.when(c + _NBUF < n_chunks)
                def _():
                    wait_idx(c + _NBUF, b)
                    clamp(b)
                    start_gather(b)
            return carry

        lax.fori_loop(0, n_chunks // _NBUF, ring_body, 0)

    return gather_kernel


@jax.jit
def kernel(x, table):
    batch, seq = x.shape
    flat = x.reshape(batch * seq).astype(jnp.int32)
    out = _make_gather(batch * seq, _EMBED_DIM, _VOCAB_SIZE)(flat, table)
    return out.reshape(batch, seq, _EMBED_DIM)
